# baseline (speedup 1.0000x reference)
import jax
import jax.numpy as jnp
from jax.experimental import pallas as pl
from jax.experimental.pallas import tpu as pltpu
from jax.experimental.pallas import tpu_sc as plsc

_B = 16384
_D = 64
_NS = 15
_NPAD = 16
_BLK = 4096
_LANE = 128
_NCORES = 1


def _sc_gather_tiles(embT, tile_idx):
    mesh = plsc.ScalarSubcoreMesh(axis_name="core", num_cores=_NCORES)
    per_core = _NPAD // _NCORES

    @pl.kernel(
        out_type=jax.ShapeDtypeStruct((_NPAD, _D, _LANE), embT.dtype),
        mesh=mesh,
        scratch_types=[
            pltpu.SMEM((1, _NPAD), jnp.int32),
            pltpu.SemaphoreType.DMA,
        ],
    )
    def gather_kernel(emb_hbm, idx_hbm, out_hbm, idx_smem, sem):
        core = jax.lax.axis_index("core")
        pltpu.async_copy(idx_hbm, idx_smem, sem).wait()
        copies = []
        for n in range(per_core):
            slot = core * per_core + n
            base = pl.multiple_of(idx_smem[0, slot] * _LANE, _LANE)
            cp = pltpu.make_async_copy(
                emb_hbm.at[:, pl.ds(base, _LANE)],
                out_hbm.at[slot],
                sem,
            )
            cp.start()
            copies.append(cp)
        for cp in copies:
            cp.wait()

    return gather_kernel(embT, tile_idx)


def _logsig(x):
    return jnp.minimum(x, 0.0) - jnp.log1p(jnp.exp(-jnp.abs(x)))


def _loss_body(centerT_ref, contextT_ref, staged_ref, col_ref, out_ref,
               s2_ref):
    i = pl.program_id(0)

    @pl.when(i == 0)
    def _():
        w = staged_ref[...]
        w = jnp.where(jnp.abs(w) < jnp.float32(1e30), w, 0.0)
        col = col_ref[...].reshape(_NPAD, 1, 1)
        j = jax.lax.broadcasted_iota(jnp.int32, (_NPAD, _D, _LANE), 2)
        s2_ref[...] = jnp.sum(
            jnp.where(j == col, w, 0.0), axis=2
        )

    c = centerT_ref[...]
    t = contextT_ref[...]
    s = s2_ref[...]

    pos = jnp.sum(c * t, axis=0, keepdims=True)
    pos_ls = _logsig(pos)

    dots = jax.lax.dot_general(
        s, c, (((1,), (0,)), ((), ())), preferred_element_type=jnp.float32
    )
    neg_ls = _logsig(-dots)
    row = jax.lax.broadcasted_iota(jnp.int32, neg_ls.shape, 0)
    neg_ls = jnp.where(row < _NS, neg_ls, 0.0)

    part = jnp.sum(pos_ls) + jnp.sum(neg_ls)

    @pl.when(i == 0)
    def _():
        out_ref[0, 0] = 0.0

    out_ref[0, 0] += -part

    @pl.when(i == (_B // _BLK) - 1)
    def _():
        out_ref[0, 0] = out_ref[0, 0] * (1.0 / _B)


def kernel(center, context, neg_idxs, emb_table):
    idx = neg_idxs.astype(jnp.int32)
    idx = jnp.concatenate([idx, idx[:1]])
    tile_idx = (idx // _LANE).reshape(1, _NPAD)
    col = (idx % _LANE).reshape(_NPAD, 1)

    cT = center.T
    tT = context.T
    eT = emb_table.T
    staged = _sc_gather_tiles(eT, tile_idx)

    nb = _B // _BLK
    out = pl.pallas_call(
        _loss_body,
        grid=(nb,),
        in_specs=[
            pl.BlockSpec((_D, _BLK), lambda i: (0, i)),
            pl.BlockSpec((_D, _BLK), lambda i: (0, i)),
            pl.BlockSpec((_NPAD, _D, _LANE), lambda i: (0, 0, 0)),
            pl.BlockSpec((_NPAD, 1), lambda i: (0, 0)),
        ],
        out_specs=pl.BlockSpec(
            (1, 1), lambda i: (0, 0), memory_space=pltpu.SMEM
        ),
        out_shape=jax.ShapeDtypeStruct((1, 1), jnp.float32),
        scratch_shapes=[pltpu.VMEM((_NPAD, _D), jnp.float32)],
    )(cT, tT, staged, col)
    return out[0, 0]


# speedup vs baseline: 1.0252x; 1.0252x over previous
import jax
import jax.numpy as jnp
from jax.experimental import pallas as pl
from jax.experimental.pallas import tpu as pltpu
from jax.experimental.pallas import tpu_sc as plsc

_B = 16384
_D = 64
_NS = 15
_NPAD = 16
_BLK = 8192
_LANE = 128
_NCORES = 1


def _sc_gather_tiles(embT, tile_idx):
    mesh = plsc.ScalarSubcoreMesh(axis_name="core", num_cores=_NCORES)
    per_core = _NPAD // _NCORES

    @pl.kernel(
        out_type=jax.ShapeDtypeStruct((_NPAD, _D, _LANE), embT.dtype),
        mesh=mesh,
        scratch_types=[
            pltpu.SMEM((1, _NPAD), jnp.int32),
            pltpu.SemaphoreType.DMA,
        ],
    )
    def gather_kernel(emb_hbm, idx_hbm, out_hbm, idx_smem, sem):
        core = jax.lax.axis_index("core")
        pltpu.async_copy(idx_hbm, idx_smem, sem).wait()
        copies = []
        for n in range(per_core):
            slot = core * per_core + n
            base = pl.multiple_of(idx_smem[0, slot] * _LANE, _LANE)
            cp = pltpu.make_async_copy(
                emb_hbm.at[:, pl.ds(base, _LANE)],
                out_hbm.at[slot],
                sem,
            )
            cp.start()
            copies.append(cp)
        for cp in copies:
            cp.wait()

    return gather_kernel(embT, tile_idx)


def _logsig(x):
    return jnp.minimum(x, 0.0) - jnp.log1p(jnp.exp(-jnp.abs(x)))


def _loss_body(centerT_ref, contextT_ref, staged_ref, col_ref, out_ref,
               s2_ref):
    i = pl.program_id(0)

    @pl.when(i == 0)
    def _():
        w = staged_ref[...]
        w = jnp.where(jnp.abs(w) < jnp.float32(1e30), w, 0.0)
        col = col_ref[...].reshape(_NPAD, 1, 1)
        j = jax.lax.broadcasted_iota(jnp.int32, (_NPAD, _D, _LANE), 2)
        s2_ref[...] = jnp.sum(
            jnp.where(j == col, w, 0.0), axis=2
        )

    c = centerT_ref[...]
    t = contextT_ref[...]
    s = s2_ref[...]

    pos = jnp.sum(c * t, axis=0, keepdims=True)
    pos_ls = _logsig(pos)

    dots = jax.lax.dot_general(
        s, c, (((1,), (0,)), ((), ())), preferred_element_type=jnp.float32
    )
    neg_ls = _logsig(-dots)
    row = jax.lax.broadcasted_iota(jnp.int32, neg_ls.shape, 0)
    neg_ls = jnp.where(row < _NS, neg_ls, 0.0)

    part = jnp.sum(pos_ls) + jnp.sum(neg_ls)

    @pl.when(i == 0)
    def _():
        out_ref[0, 0] = 0.0

    out_ref[0, 0] += -part

    @pl.when(i == (_B // _BLK) - 1)
    def _():
        out_ref[0, 0] = out_ref[0, 0] * (1.0 / _B)


def kernel(center, context, neg_idxs, emb_table):
    idx = neg_idxs.astype(jnp.int32)
    idx = jnp.concatenate([idx, idx[:1]])
    tile_idx = (idx // _LANE).reshape(1, _NPAD)
    col = (idx % _LANE).reshape(_NPAD, 1)

    cT = center.T
    tT = context.T
    eT = emb_table.T
    staged = _sc_gather_tiles(eT, tile_idx)

    nb = _B // _BLK
    out = pl.pallas_call(
        _loss_body,
        grid=(nb,),
        in_specs=[
            pl.BlockSpec((_D, _BLK), lambda i: (0, i)),
            pl.BlockSpec((_D, _BLK), lambda i: (0, i)),
            pl.BlockSpec((_NPAD, _D, _LANE), lambda i: (0, 0, 0)),
            pl.BlockSpec((_NPAD, 1), lambda i: (0, 0)),
        ],
        out_specs=pl.BlockSpec(
            (1, 1), lambda i: (0, 0), memory_space=pltpu.SMEM
        ),
        out_shape=jax.ShapeDtypeStruct((1, 1), jnp.float32),
        scratch_shapes=[pltpu.VMEM((_NPAD, _D), jnp.float32)],
    )(cT, tT, staged, col)
    return out[0, 0]
